# baseline (device time: 13949 ns/iter reference)
import jax
import jax.numpy as jnp
from jax import lax
from jax.experimental import pallas as pl
from jax.experimental.pallas import tpu as pltpu

N_DEV = 16
CHUNK_M = 256


def kernel(x):
    x = pltpu.with_memory_space_constraint(x, pltpu.MemorySpace.HBM)
    m_per, n = x.shape
    n_chunks = m_per // CHUNK_M
    inv_m = 1.0 / float(N_DEV * m_per)

    def body(
        x_hbm_ref,
        out_ref,
        partial_ref,
        buf_ref,
        comm_ref,
        copy_sems,
        send_sems,
        recv_sems,
    ):
        my_pos = lax.axis_index("i")

        barrier_sem = pltpu.get_barrier_semaphore()
        for d in range(1, N_DEV):
            pl.semaphore_signal(
                barrier_sem,
                inc=1,
                device_id=((my_pos + d) % N_DEV,),
                device_id_type=pl.DeviceIdType.MESH,
            )

        def chunk_copy(i, slot):
            return pltpu.make_async_copy(
                x_hbm_ref.at[pl.ds(i * CHUNK_M, CHUNK_M), :],
                buf_ref.at[slot],
                copy_sems.at[slot],
            )

        chunk_copy(0, 0).start()
        chunk_copy(1, 1).start()
        acc = jnp.zeros((1, n), jnp.float32)
        for i in range(n_chunks):
            slot = i % 3
            chunk_copy(i, slot).wait()
            if i + 2 < n_chunks:
                chunk_copy(i + 2, (i + 2) % 3).start()
            acc = acc + jnp.sum(
                buf_ref[slot].astype(jnp.float32), axis=0, keepdims=True
            )
        partial_ref[...] = acc

        pl.semaphore_wait(barrier_sem, N_DEV - 1)

        rdmas = []
        for d in range(1, N_DEV):
            rdma = pltpu.make_async_remote_copy(
                src_ref=partial_ref,
                dst_ref=comm_ref.at[d - 1],
                send_sem=send_sems.at[d - 1],
                recv_sem=recv_sems.at[d - 1],
                device_id=((my_pos + d) % N_DEV,),
                device_id_type=pl.DeviceIdType.MESH,
            )
            rdma.start()
            rdmas.append(rdma)

        total = partial_ref[...]
        for d in range(1, N_DEV):
            rdmas[d - 1].wait_recv()
            total = total + comm_ref[d - 1]
        out_ref[...] = total * inv_m

        for d in range(1, N_DEV):
            rdmas[d - 1].wait_send()

    return pl.pallas_call(
        body,
        out_shape=jax.ShapeDtypeStruct((1, n), jnp.float32),
        in_specs=[pl.BlockSpec(memory_space=pl.ANY)],
        out_specs=pl.BlockSpec(memory_space=pltpu.VMEM),
        scratch_shapes=[
            pltpu.VMEM((1, n), jnp.float32),
            pltpu.VMEM((3, CHUNK_M, n), x.dtype),
            pltpu.VMEM((N_DEV - 1, 1, n), jnp.float32),
            pltpu.SemaphoreType.DMA((3,)),
            pltpu.SemaphoreType.DMA((N_DEV - 1,)),
            pltpu.SemaphoreType.DMA((N_DEV - 1,)),
        ],
        compiler_params=pltpu.CompilerParams(collective_id=0),
    )(x)


# device time: 11908 ns/iter; 1.1714x vs baseline; 1.1714x over previous
import jax
import jax.numpy as jnp
from jax import lax
from jax.experimental import pallas as pl
from jax.experimental.pallas import tpu as pltpu

N_DEV = 16
CHUNK_M = 256


def kernel(x):
    x = pltpu.with_memory_space_constraint(x, pltpu.MemorySpace.HBM)
    m_per, n = x.shape
    n_chunks = m_per // CHUNK_M
    inv_m = 1.0 / float(N_DEV * m_per)

    def body(
        x_hbm_ref,
        out_ref,
        partial_ref,
        buf_ref,
        comm_ref,
        copy_sems,
        send_sems,
        recv_sems,
    ):
        my_pos = lax.axis_index("i")

        barrier_sem = pltpu.get_barrier_semaphore()
        for d in range(1, N_DEV):
            pl.semaphore_signal(
                barrier_sem,
                inc=1,
                device_id=((my_pos + d) % N_DEV,),
                device_id_type=pl.DeviceIdType.MESH,
            )

        def chunk_copy(i, slot):
            return pltpu.make_async_copy(
                x_hbm_ref.at[pl.ds(i * CHUNK_M, CHUNK_M), :],
                buf_ref.at[slot],
                copy_sems.at[slot],
            )

        for i in range(n_chunks):
            chunk_copy(i, i).start()
        acc = jnp.zeros((1, n), jnp.float32)
        for i in range(n_chunks):
            chunk_copy(i, i).wait()
            acc = acc + jnp.sum(
                buf_ref[i].astype(jnp.float32), axis=0, keepdims=True
            )
        partial_ref[...] = acc

        pl.semaphore_wait(barrier_sem, N_DEV - 1)

        rdmas = []
        for d in range(1, N_DEV):
            rdma = pltpu.make_async_remote_copy(
                src_ref=partial_ref,
                dst_ref=comm_ref.at[d - 1],
                send_sem=send_sems.at[d - 1],
                recv_sem=recv_sems.at[d - 1],
                device_id=((my_pos + d) % N_DEV,),
                device_id_type=pl.DeviceIdType.MESH,
            )
            rdma.start()
            rdmas.append(rdma)

        total = partial_ref[...]
        for d in range(1, N_DEV):
            rdmas[d - 1].wait_recv()
            total = total + comm_ref[d - 1]
        out_ref[...] = total * inv_m

        for d in range(1, N_DEV):
            rdmas[d - 1].wait_send()

    return pl.pallas_call(
        body,
        out_shape=jax.ShapeDtypeStruct((1, n), jnp.float32),
        in_specs=[pl.BlockSpec(memory_space=pl.ANY)],
        out_specs=pl.BlockSpec(memory_space=pltpu.VMEM),
        scratch_shapes=[
            pltpu.VMEM((1, n), jnp.float32),
            pltpu.VMEM((m_per // CHUNK_M, CHUNK_M, n), x.dtype),
            pltpu.VMEM((N_DEV - 1, 1, n), jnp.float32),
            pltpu.SemaphoreType.DMA((m_per // CHUNK_M,)),
            pltpu.SemaphoreType.DMA((N_DEV - 1,)),
            pltpu.SemaphoreType.DMA((N_DEV - 1,)),
        ],
        compiler_params=pltpu.CompilerParams(collective_id=0),
    )(x)
